# baseline (speedup 1.0000x reference)
import jax
import jax.numpy as jnp
from jax.experimental import pallas as pl
from jax.experimental.pallas import tpu as pltpu

_K = 1024
_C = 256
_S = 8


def _vq_kernel(z_ref, w_ref, out_ref, wm2_ref, wnorm_ref):
    @pl.when((pl.program_id(0) + pl.program_id(1)) == 0)
    def _prep():
        wmat = w_ref[...]
        wm2_ref[...] = wmat * jnp.float32(-2.0)
        wnorm_ref[...] = jnp.sum(wmat * wmat, axis=1, keepdims=True)

    zb = z_ref[0]
    cross2 = jax.lax.dot(wm2_ref[...], zb, preferred_element_type=jnp.float32)
    xnorm = jnp.sum(zb * zb, axis=0, keepdims=True)

    def chunk_scores(c):
        sl = slice(c * _S, (c + 1) * _S)
        return (xnorm + cross2[sl]) + wnorm_ref[sl]

    mv = chunk_scores(0)
    mi = jnp.zeros(mv.shape, jnp.float32)
    for c in range(1, _K // _S):
        sc = chunk_scores(c)
        upd = sc < mv
        mv = jnp.minimum(sc, mv)
        mi = jnp.where(upd, jnp.float32(c), mi)

    m = jnp.min(mv, axis=0, keepdims=True)
    sub = jax.lax.broadcasted_iota(jnp.int32, mv.shape, 0)
    gid = mi.astype(jnp.int32) * _S + sub
    hit = jnp.where(mv == m, gid, _K)
    out_ref[0, 0] = jnp.min(hit, axis=0).astype(jnp.int32)


def kernel(z_e_x, W):
    b, c, h, w = z_e_x.shape
    hw = h * w
    z = z_e_x.reshape(b, c, hw)
    out = pl.pallas_call(
        _vq_kernel,
        grid=(b, 2),
        in_specs=[
            pl.BlockSpec((1, c, hw // 2), lambda i, j: (i, 0, j)),
            pl.BlockSpec((_K, _C), lambda i, j: (0, 0)),
        ],
        out_specs=pl.BlockSpec((1, 1, hw // 2), lambda i, j: (i, 0, j)),
        out_shape=jax.ShapeDtypeStruct((b, 1, hw), jnp.int32),
        scratch_shapes=[
            pltpu.VMEM((_K, _C), jnp.float32),
            pltpu.VMEM((_K, 1), jnp.float32),
        ],
    )(z, W)
    return out.reshape(b, h, w)


# speedup vs baseline: 1.1629x; 1.1629x over previous
import jax
import jax.numpy as jnp
from jax.experimental import pallas as pl
from jax.experimental.pallas import tpu as pltpu

_K = 1024
_C = 256
_S = 8


def _vq_kernel(z_ref, w_ref, out_ref, wnorm_ref):
    @pl.when(pl.program_id(0) == 0)
    def _prep():
        wmat = w_ref[...]
        wnorm_ref[...] = jnp.sum(wmat * wmat, axis=1, keepdims=True)

    zbm2 = z_ref[0] * jnp.float32(-2.0)
    cross2 = jax.lax.dot(w_ref[...], zbm2, preferred_element_type=jnp.float32)
    xnorm = jnp.sum(zbm2 * zbm2, axis=0, keepdims=True) * jnp.float32(0.25)

    def chunk_scores(c):
        sl = slice(c * _S, (c + 1) * _S)
        return (xnorm + cross2[sl]) + wnorm_ref[sl]

    mv = chunk_scores(0)
    mi = jnp.zeros(mv.shape, jnp.float32)
    for c in range(1, _K // _S):
        sc = chunk_scores(c)
        upd = sc < mv
        mv = jnp.minimum(sc, mv)
        mi = jnp.where(upd, jnp.float32(c), mi)

    m = jnp.min(mv, axis=0, keepdims=True)
    sub = jax.lax.broadcasted_iota(jnp.int32, mv.shape, 0)
    gid = mi.astype(jnp.int32) * _S + sub
    hit = jnp.where(mv == m, gid, _K)
    out_ref[0, 0] = jnp.min(hit, axis=0).astype(jnp.int32)


def kernel(z_e_x, W):
    b, c, h, w = z_e_x.shape
    hw = h * w
    z = z_e_x.reshape(b, c, hw)
    out = pl.pallas_call(
        _vq_kernel,
        grid=(b,),
        in_specs=[
            pl.BlockSpec((1, c, hw), lambda i: (i, 0, 0)),
            pl.BlockSpec((_K, _C), lambda i: (0, 0)),
        ],
        out_specs=pl.BlockSpec((1, 1, hw), lambda i: (i, 0, 0)),
        out_shape=jax.ShapeDtypeStruct((b, 1, hw), jnp.int32),
        scratch_shapes=[
            pltpu.VMEM((_K, 1), jnp.float32),
        ],
    )(z, W)
    return out.reshape(b, h, w)


# speedup vs baseline: 1.1879x; 1.0215x over previous
import jax
import jax.numpy as jnp
from jax.experimental import pallas as pl
from jax.experimental.pallas import tpu as pltpu

_K = 1024
_C = 256
_S = 8


def _vq_kernel(z_ref, w_ref, out_ref, wm2_ref, wnorm_ref):
    @pl.when(pl.program_id(0) == 0)
    def _prep():
        wmat = w_ref[...]
        wm2_ref[...] = wmat * jnp.float32(-2.0)
        wnorm_ref[...] = jnp.sum(wmat * wmat, axis=1, keepdims=True)

    zb = z_ref[0]
    cross2 = jax.lax.dot(wm2_ref[...], zb, preferred_element_type=jnp.float32)
    xnorm = jnp.sum(zb * zb, axis=0, keepdims=True)

    def chunk_scores(c):
        sl = slice(c * _S, (c + 1) * _S)
        return (xnorm + cross2[sl]) + wnorm_ref[sl]

    mv = chunk_scores(0)
    mi = jnp.zeros(mv.shape, jnp.float32)
    for c in range(1, _K // _S):
        sc = chunk_scores(c)
        upd = sc < mv
        mv = jnp.minimum(sc, mv)
        mi = jnp.where(upd, jnp.float32(c), mi)

    m = jnp.min(mv, axis=0, keepdims=True)
    sub = jax.lax.broadcasted_iota(jnp.int32, mv.shape, 0)
    gid = mi.astype(jnp.int32) * _S + sub
    hit = jnp.where(mv == m, gid, _K)
    out_ref[0, 0] = jnp.min(hit, axis=0).astype(jnp.int32)


def kernel(z_e_x, W):
    b, c, h, w = z_e_x.shape
    hw = h * w
    z = z_e_x.reshape(b, c, hw)
    out = pl.pallas_call(
        _vq_kernel,
        grid=(b,),
        in_specs=[
            pl.BlockSpec((1, c, hw), lambda i: (i, 0, 0)),
            pl.BlockSpec((_K, _C), lambda i: (0, 0)),
        ],
        out_specs=pl.BlockSpec((1, 1, hw), lambda i: (i, 0, 0)),
        out_shape=jax.ShapeDtypeStruct((b, 1, hw), jnp.int32),
        scratch_shapes=[
            pltpu.VMEM((_K, _C), jnp.float32),
            pltpu.VMEM((_K, 1), jnp.float32),
        ],
    )(z, W)
    return out.reshape(b, h, w)
